# baseline (device time: 39776 ns/iter reference)
import jax
import jax.numpy as jnp
from jax import lax
from jax.experimental import pallas as pl
from jax.experimental.pallas import tpu as pltpu

N_DEV = 32
P = 8
G = 4


def kernel(dy, W):
    m, _ = dy.shape
    n = W.shape[0]
    h = m // 2
    ba = h // P
    bb = h // G
    f = bb // P

    def body(dy_ref, w_ref, out_ref, p_ref,
             rsA1, rsB1, redA, redB, rsA2, rsB2, agB2,
             sA1, rA1, sB1, rB1, sA2, rA2, sB2, rB2,
             sB3, rB3, sA4, rA4, sB4, rB4):
        me = lax.axis_index("i")
        g = lax.div(me, P)
        r = lax.rem(me, P)

        barrier_sem = pltpu.get_barrier_semaphore()
        for off in range(1, P):
            pl.semaphore_signal(
                barrier_sem, inc=1,
                device_id=g * P + lax.rem(r + off, P),
                device_id_type=pl.DeviceIdType.LOGICAL,
            )
        for off in range(1, G):
            pl.semaphore_signal(
                barrier_sem, inc=1,
                device_id=lax.rem(g + off, G) * P + r,
                device_id_type=pl.DeviceIdType.LOGICAL,
            )
        pl.semaphore_wait(barrier_sem, (P - 1) + (G - 1))

        p_ref[:, :] = lax.dot_general(
            dy_ref[:, :], w_ref[:, :],
            dimension_numbers=(((1,), (1,)), ((), ())),
            preferred_element_type=jnp.float32,
        )

        sends = []

        def send(src, dst, send_sem, recv_sem, dev):
            rdma = pltpu.make_async_remote_copy(
                src_ref=src, dst_ref=dst, send_sem=send_sem,
                recv_sem=recv_sem, device_id=dev,
                device_id_type=pl.DeviceIdType.LOGICAL,
            )
            rdma.start()
            sends.append(rdma)

        def wait_recv(dst, recv_sem):
            pltpu.make_async_remote_copy(
                src_ref=dst, dst_ref=dst, send_sem=recv_sem,
                recv_sem=recv_sem, device_id=me,
                device_id_type=pl.DeviceIdType.LOGICAL,
            ).wait_recv()

        for off in range(1, P):
            jr = lax.rem(r + off, P)
            send(p_ref.at[pl.ds(jr * ba, ba), :], rsA1.at[r],
                 sA1.at[off], rA1.at[r], g * P + jr)
        for off in range(1, G):
            qg = lax.rem(g + off, G)
            send(p_ref.at[pl.ds(h + qg * bb, bb), :], rsB1.at[g],
                 sB1.at[off], rB1.at[g], qg * P + r)
        rsA1[pl.ds(r, 1), :, :] = p_ref[pl.ds(r * ba, ba), :].reshape(
            1, ba, n)
        rsB1[pl.ds(g, 1), :, :] = p_ref[pl.ds(h + g * bb, bb), :].reshape(
            1, bb, n)

        for off in range(1, P):
            s = lax.rem(r + off, P)
            wait_recv(rsA1.at[s], rA1.at[s])
        redA[:, :] = jnp.sum(rsA1[:, :, :], axis=0)
        for off in range(1, G):
            qg = lax.rem(g + off, G)
            send(redA, rsA2.at[g], sA2.at[off], rA2.at[g], qg * P + r)
        rsA2[pl.ds(g, 1), :, :] = redA[:, :].reshape(1, ba, n)

        for off in range(1, G):
            s = lax.rem(g + off, G)
            wait_recv(rsB1.at[s], rB1.at[s])
        redB[:, :] = jnp.sum(rsB1[:, :, :], axis=0)
        for off in range(1, P):
            jr = lax.rem(r + off, P)
            send(redB.at[pl.ds(jr * f, f), :], rsB2.at[r],
                 sB2.at[off], rB2.at[r], g * P + jr)
        rsB2[pl.ds(r, 1), :, :] = redB[pl.ds(r * f, f), :].reshape(1, f, n)

        for off in range(1, G):
            s = lax.rem(g + off, G)
            wait_recv(rsA2.at[s], rA2.at[s])
        out_ref[pl.ds(r * ba, ba), :] = jnp.sum(rsA2[:, :, :], axis=0)
        for off in range(1, P):
            jr = lax.rem(r + off, P)
            send(out_ref.at[pl.ds(r * ba, ba), :],
                 out_ref.at[pl.ds(r * ba, ba), :],
                 sA4.at[off], rA4.at[r], g * P + jr)

        for off in range(1, P):
            s = lax.rem(r + off, P)
            wait_recv(rsB2.at[s], rB2.at[s])
        agB2[pl.ds(r, 1), :, :] = jnp.sum(rsB2[:, :, :], axis=0,
                                          keepdims=True)
        for off in range(1, P):
            jr = lax.rem(r + off, P)
            send(agB2.at[r], agB2.at[r], sB3.at[off], rB3.at[r],
                 g * P + jr)
        for off in range(1, P):
            s = lax.rem(r + off, P)
            wait_recv(agB2.at[s], rB3.at[s])
        out_ref[pl.ds(h + g * bb, bb), :] = agB2[:, :, :].reshape(bb, n)

        for off in range(1, G):
            qg = lax.rem(g + off, G)
            send(out_ref.at[pl.ds(h + g * bb, bb), :],
                 out_ref.at[pl.ds(h + g * bb, bb), :],
                 sB4.at[off], rB4.at[g], qg * P + r)

        for off in range(1, P):
            s = lax.rem(r + off, P)
            wait_recv(out_ref.at[pl.ds(s * ba, ba), :], rA4.at[s])
        for off in range(1, G):
            s = lax.rem(g + off, G)
            wait_recv(out_ref.at[pl.ds(h + s * bb, bb), :], rB4.at[s])

        for rdma in sends:
            rdma.wait_send()

    return pl.pallas_call(
        body,
        out_shape=jax.ShapeDtypeStruct((m, n), jnp.float32),
        in_specs=[
            pl.BlockSpec(memory_space=pltpu.VMEM),
            pl.BlockSpec(memory_space=pltpu.VMEM),
        ],
        out_specs=pl.BlockSpec(memory_space=pltpu.VMEM),
        scratch_shapes=[
            pltpu.VMEM((m, n), jnp.float32),
            pltpu.VMEM((P, ba, n), jnp.float32),
            pltpu.VMEM((G, bb, n), jnp.float32),
            pltpu.VMEM((ba, n), jnp.float32),
            pltpu.VMEM((bb, n), jnp.float32),
            pltpu.VMEM((G, ba, n), jnp.float32),
            pltpu.VMEM((P, f, n), jnp.float32),
            pltpu.VMEM((P, f, n), jnp.float32),
            pltpu.SemaphoreType.DMA((P,)),
            pltpu.SemaphoreType.DMA((P,)),
            pltpu.SemaphoreType.DMA((G,)),
            pltpu.SemaphoreType.DMA((G,)),
            pltpu.SemaphoreType.DMA((G,)),
            pltpu.SemaphoreType.DMA((G,)),
            pltpu.SemaphoreType.DMA((P,)),
            pltpu.SemaphoreType.DMA((P,)),
            pltpu.SemaphoreType.DMA((P,)),
            pltpu.SemaphoreType.DMA((P,)),
            pltpu.SemaphoreType.DMA((P,)),
            pltpu.SemaphoreType.DMA((P,)),
            pltpu.SemaphoreType.DMA((G,)),
            pltpu.SemaphoreType.DMA((G,)),
        ],
        compiler_params=pltpu.CompilerParams(collective_id=0),
    )(dy, W)


# device time: 34637 ns/iter; 1.1484x vs baseline; 1.1484x over previous
import jax
import jax.numpy as jnp
from jax import lax
from jax.experimental import pallas as pl
from jax.experimental.pallas import tpu as pltpu

N_DEV = 32
P = 8
G = 4


def kernel(dy, W):
    m, _ = dy.shape
    n = W.shape[0]
    h = m // 2
    ba = h // P
    bb = h // G
    f = ba // G

    def body(dy_ref, w_ref, out_ref, p_ref,
             rsA1, rsB1, redA, redB, rsA2, rsB2, agA2, agB2,
             sA1, rA1, sB1, rB1, sA2, rA2, sB2, rB2,
             sA3, rA3, sB3, rB3, sA4, rA4, sB4, rB4):
        me = lax.axis_index("i")
        g = lax.div(me, P)
        r = lax.rem(me, P)

        barrier_sem = pltpu.get_barrier_semaphore()
        for off in range(1, P):
            pl.semaphore_signal(
                barrier_sem, inc=1,
                device_id=g * P + lax.rem(r + off, P),
                device_id_type=pl.DeviceIdType.LOGICAL,
            )
        for off in range(1, G):
            pl.semaphore_signal(
                barrier_sem, inc=1,
                device_id=lax.rem(g + off, G) * P + r,
                device_id_type=pl.DeviceIdType.LOGICAL,
            )
        pl.semaphore_wait(barrier_sem, (P - 1) + (G - 1))

        p_ref[:, :] = lax.dot_general(
            dy_ref[:, :], w_ref[:, :],
            dimension_numbers=(((1,), (1,)), ((), ())),
            preferred_element_type=jnp.float32,
        )

        sends = []

        def send(src, dst, send_sem, recv_sem, dev):
            rdma = pltpu.make_async_remote_copy(
                src_ref=src, dst_ref=dst, send_sem=send_sem,
                recv_sem=recv_sem, device_id=dev,
                device_id_type=pl.DeviceIdType.LOGICAL,
            )
            rdma.start()
            sends.append(rdma)

        def wait_recv(dst, recv_sem):
            pltpu.make_async_remote_copy(
                src_ref=dst, dst_ref=dst, send_sem=recv_sem,
                recv_sem=recv_sem, device_id=me,
                device_id_type=pl.DeviceIdType.LOGICAL,
            ).wait_recv()

        for off in range(1, P):
            jr = lax.rem(r + off, P)
            send(p_ref.at[pl.ds(jr * ba, ba), :], rsA1.at[r],
                 sA1.at[off], rA1.at[r], g * P + jr)
        for off in range(1, G):
            qg = lax.rem(g + off, G)
            send(p_ref.at[pl.ds(h + qg * bb, bb), :], rsB1.at[g],
                 sB1.at[off], rB1.at[g], qg * P + r)
        rsA1[pl.ds(r, 1), :, :] = p_ref[pl.ds(r * ba, ba), :].reshape(
            1, ba, n)
        rsB1[pl.ds(g, 1), :, :] = p_ref[pl.ds(h + g * bb, bb), :].reshape(
            1, bb, n)
        for off in range(1, P):
            s = lax.rem(r + off, P)
            wait_recv(rsA1.at[s], rA1.at[s])
        for off in range(1, G):
            s = lax.rem(g + off, G)
            wait_recv(rsB1.at[s], rB1.at[s])
        redA[:, :] = jnp.sum(rsA1[:, :, :], axis=0)
        redB[:, :] = jnp.sum(rsB1[:, :, :], axis=0)

        for off in range(1, G):
            qg = lax.rem(g + off, G)
            send(redA.at[pl.ds(qg * f, f), :], rsA2.at[g],
                 sA2.at[off], rA2.at[g], qg * P + r)
        for off in range(1, P):
            jr = lax.rem(r + off, P)
            send(redB.at[pl.ds(jr * f, f), :], rsB2.at[r],
                 sB2.at[off], rB2.at[r], g * P + jr)
        rsA2[pl.ds(g, 1), :, :] = redA[pl.ds(g * f, f), :].reshape(1, f, n)
        rsB2[pl.ds(r, 1), :, :] = redB[pl.ds(r * f, f), :].reshape(1, f, n)
        for off in range(1, G):
            s = lax.rem(g + off, G)
            wait_recv(rsA2.at[s], rA2.at[s])
        for off in range(1, P):
            s = lax.rem(r + off, P)
            wait_recv(rsB2.at[s], rB2.at[s])
        agA2[pl.ds(g, 1), :, :] = jnp.sum(rsA2[:, :, :], axis=0,
                                          keepdims=True)
        agB2[pl.ds(r, 1), :, :] = jnp.sum(rsB2[:, :, :], axis=0,
                                          keepdims=True)

        for off in range(1, G):
            qg = lax.rem(g + off, G)
            send(agA2.at[g], agA2.at[g], sA3.at[off], rA3.at[g],
                 qg * P + r)
        for off in range(1, P):
            jr = lax.rem(r + off, P)
            send(agB2.at[r], agB2.at[r], sB3.at[off], rB3.at[r],
                 g * P + jr)
        for off in range(1, G):
            s = lax.rem(g + off, G)
            wait_recv(agA2.at[s], rA3.at[s])
        for off in range(1, P):
            s = lax.rem(r + off, P)
            wait_recv(agB2.at[s], rB3.at[s])
        out_ref[pl.ds(r * ba, ba), :] = agA2[:, :, :].reshape(ba, n)
        out_ref[pl.ds(h + g * bb, bb), :] = agB2[:, :, :].reshape(bb, n)

        for off in range(1, P):
            jr = lax.rem(r + off, P)
            send(out_ref.at[pl.ds(r * ba, ba), :],
                 out_ref.at[pl.ds(r * ba, ba), :],
                 sA4.at[off], rA4.at[r], g * P + jr)
        for off in range(1, G):
            qg = lax.rem(g + off, G)
            send(out_ref.at[pl.ds(h + g * bb, bb), :],
                 out_ref.at[pl.ds(h + g * bb, bb), :],
                 sB4.at[off], rB4.at[g], qg * P + r)
        for off in range(1, P):
            s = lax.rem(r + off, P)
            wait_recv(out_ref.at[pl.ds(s * ba, ba), :], rA4.at[s])
        for off in range(1, G):
            s = lax.rem(g + off, G)
            wait_recv(out_ref.at[pl.ds(h + s * bb, bb), :], rB4.at[s])

        for rdma in sends:
            rdma.wait_send()

    return pl.pallas_call(
        body,
        out_shape=jax.ShapeDtypeStruct((m, n), jnp.float32),
        in_specs=[
            pl.BlockSpec(memory_space=pltpu.VMEM),
            pl.BlockSpec(memory_space=pltpu.VMEM),
        ],
        out_specs=pl.BlockSpec(memory_space=pltpu.VMEM),
        scratch_shapes=[
            pltpu.VMEM((m, n), jnp.float32),
            pltpu.VMEM((P, ba, n), jnp.float32),
            pltpu.VMEM((G, bb, n), jnp.float32),
            pltpu.VMEM((ba, n), jnp.float32),
            pltpu.VMEM((bb, n), jnp.float32),
            pltpu.VMEM((G, f, n), jnp.float32),
            pltpu.VMEM((P, f, n), jnp.float32),
            pltpu.VMEM((G, f, n), jnp.float32),
            pltpu.VMEM((P, f, n), jnp.float32),
            pltpu.SemaphoreType.DMA((P,)),
            pltpu.SemaphoreType.DMA((P,)),
            pltpu.SemaphoreType.DMA((G,)),
            pltpu.SemaphoreType.DMA((G,)),
            pltpu.SemaphoreType.DMA((G,)),
            pltpu.SemaphoreType.DMA((G,)),
            pltpu.SemaphoreType.DMA((P,)),
            pltpu.SemaphoreType.DMA((P,)),
            pltpu.SemaphoreType.DMA((G,)),
            pltpu.SemaphoreType.DMA((G,)),
            pltpu.SemaphoreType.DMA((P,)),
            pltpu.SemaphoreType.DMA((P,)),
            pltpu.SemaphoreType.DMA((P,)),
            pltpu.SemaphoreType.DMA((P,)),
            pltpu.SemaphoreType.DMA((G,)),
            pltpu.SemaphoreType.DMA((G,)),
        ],
        compiler_params=pltpu.CompilerParams(collective_id=0),
    )(dy, W)


# device time: 34530 ns/iter; 1.1519x vs baseline; 1.0031x over previous
import jax
import jax.numpy as jnp
from jax import lax
from jax.experimental import pallas as pl
from jax.experimental.pallas import tpu as pltpu

N_DEV = 32
P = 8
G = 4


def kernel(dy, W):
    m, _ = dy.shape
    n = W.shape[0]
    h = m // 2
    ba = h // P
    bb = h // G
    f = ba // G

    def body(dy_ref, w_ref, out_ref, p_ref,
             rsA1, rsB1, redA, redB, rsA2, rsB2, agA2, agB2,
             sA1, rA1, sB1, rB1, sA2, rA2, sB2, rB2,
             sA3, rA3, sB3, rB3, sA4, rA4, sB4, rB4):
        me = lax.axis_index("i")
        g = lax.div(me, P)
        r = lax.rem(me, P)

        barrier_sem = pltpu.get_barrier_semaphore()
        for off in range(1, P):
            pl.semaphore_signal(
                barrier_sem, inc=1,
                device_id=g * P + lax.rem(r + off, P),
                device_id_type=pl.DeviceIdType.LOGICAL,
            )
        for off in range(1, G):
            pl.semaphore_signal(
                barrier_sem, inc=1,
                device_id=lax.rem(g + off, G) * P + r,
                device_id_type=pl.DeviceIdType.LOGICAL,
            )
        pl.semaphore_wait(barrier_sem, (P - 1) + (G - 1))

        p_ref[:, :] = lax.dot_general(
            dy_ref[:, :], w_ref[:, :],
            dimension_numbers=(((1,), (1,)), ((), ())),
            preferred_element_type=jnp.float32,
        )

        sends = []

        def send(src, dst, send_sem, recv_sem, dev):
            rdma = pltpu.make_async_remote_copy(
                src_ref=src, dst_ref=dst, send_sem=send_sem,
                recv_sem=recv_sem, device_id=dev,
                device_id_type=pl.DeviceIdType.LOGICAL,
            )
            rdma.start()
            sends.append(rdma)

        def wait_recv(dst, recv_sem):
            pltpu.make_async_remote_copy(
                src_ref=dst, dst_ref=dst, send_sem=recv_sem,
                recv_sem=recv_sem, device_id=me,
                device_id_type=pl.DeviceIdType.LOGICAL,
            ).wait_recv()

        for off in range(1, P):
            jr = lax.rem(r + off, P)
            send(p_ref.at[pl.ds(jr * ba, ba), :], rsA1.at[r],
                 sA1.at[off], rA1.at[r], g * P + jr)
        for off in range(1, G):
            qg = lax.rem(g + off, G)
            send(p_ref.at[pl.ds(h + qg * bb, bb), :], rsB1.at[g],
                 sB1.at[off], rB1.at[g], qg * P + r)
        redA[:, :] = p_ref[pl.ds(r * ba, ba), :]
        redB[:, :] = p_ref[pl.ds(h + g * bb, bb), :]
        for off in range(1, P):
            s = lax.rem(r + off, P)
            wait_recv(rsA1.at[s], rA1.at[s])
            redA[:, :] = redA[:, :] + rsA1[s, :, :]
        for off in range(1, G):
            s = lax.rem(g + off, G)
            wait_recv(rsB1.at[s], rB1.at[s])
            redB[:, :] = redB[:, :] + rsB1[s, :, :]

        for off in range(1, G):
            qg = lax.rem(g + off, G)
            send(redA.at[pl.ds(qg * f, f), :], rsA2.at[g],
                 sA2.at[off], rA2.at[g], qg * P + r)
        for off in range(1, P):
            jr = lax.rem(r + off, P)
            send(redB.at[pl.ds(jr * f, f), :], rsB2.at[r],
                 sB2.at[off], rB2.at[r], g * P + jr)
        rsA2[pl.ds(g, 1), :, :] = redA[pl.ds(g * f, f), :].reshape(1, f, n)
        rsB2[pl.ds(r, 1), :, :] = redB[pl.ds(r * f, f), :].reshape(1, f, n)
        for off in range(1, G):
            s = lax.rem(g + off, G)
            wait_recv(rsA2.at[s], rA2.at[s])
        for off in range(1, P):
            s = lax.rem(r + off, P)
            wait_recv(rsB2.at[s], rB2.at[s])
        agA2[pl.ds(g, 1), :, :] = jnp.sum(rsA2[:, :, :], axis=0,
                                          keepdims=True)
        agB2[pl.ds(r, 1), :, :] = jnp.sum(rsB2[:, :, :], axis=0,
                                          keepdims=True)

        for off in range(1, G):
            qg = lax.rem(g + off, G)
            send(agA2.at[g], agA2.at[g], sA3.at[off], rA3.at[g],
                 qg * P + r)
        for off in range(1, P):
            jr = lax.rem(r + off, P)
            send(agB2.at[r], agB2.at[r], sB3.at[off], rB3.at[r],
                 g * P + jr)
        for off in range(1, G):
            s = lax.rem(g + off, G)
            wait_recv(agA2.at[s], rA3.at[s])
        for off in range(1, P):
            s = lax.rem(r + off, P)
            wait_recv(agB2.at[s], rB3.at[s])
        out_ref[pl.ds(r * ba, ba), :] = agA2[:, :, :].reshape(ba, n)
        out_ref[pl.ds(h + g * bb, bb), :] = agB2[:, :, :].reshape(bb, n)

        for off in range(1, P):
            jr = lax.rem(r + off, P)
            send(out_ref.at[pl.ds(r * ba, ba), :],
                 out_ref.at[pl.ds(r * ba, ba), :],
                 sA4.at[off], rA4.at[r], g * P + jr)
        for off in range(1, G):
            qg = lax.rem(g + off, G)
            send(out_ref.at[pl.ds(h + g * bb, bb), :],
                 out_ref.at[pl.ds(h + g * bb, bb), :],
                 sB4.at[off], rB4.at[g], qg * P + r)
        for off in range(1, P):
            s = lax.rem(r + off, P)
            wait_recv(out_ref.at[pl.ds(s * ba, ba), :], rA4.at[s])
        for off in range(1, G):
            s = lax.rem(g + off, G)
            wait_recv(out_ref.at[pl.ds(h + s * bb, bb), :], rB4.at[s])

        for rdma in sends:
            rdma.wait_send()

    return pl.pallas_call(
        body,
        out_shape=jax.ShapeDtypeStruct((m, n), jnp.float32),
        in_specs=[
            pl.BlockSpec(memory_space=pltpu.VMEM),
            pl.BlockSpec(memory_space=pltpu.VMEM),
        ],
        out_specs=pl.BlockSpec(memory_space=pltpu.VMEM),
        scratch_shapes=[
            pltpu.VMEM((m, n), jnp.float32),
            pltpu.VMEM((P, ba, n), jnp.float32),
            pltpu.VMEM((G, bb, n), jnp.float32),
            pltpu.VMEM((ba, n), jnp.float32),
            pltpu.VMEM((bb, n), jnp.float32),
            pltpu.VMEM((G, f, n), jnp.float32),
            pltpu.VMEM((P, f, n), jnp.float32),
            pltpu.VMEM((G, f, n), jnp.float32),
            pltpu.VMEM((P, f, n), jnp.float32),
            pltpu.SemaphoreType.DMA((P,)),
            pltpu.SemaphoreType.DMA((P,)),
            pltpu.SemaphoreType.DMA((G,)),
            pltpu.SemaphoreType.DMA((G,)),
            pltpu.SemaphoreType.DMA((G,)),
            pltpu.SemaphoreType.DMA((G,)),
            pltpu.SemaphoreType.DMA((P,)),
            pltpu.SemaphoreType.DMA((P,)),
            pltpu.SemaphoreType.DMA((G,)),
            pltpu.SemaphoreType.DMA((G,)),
            pltpu.SemaphoreType.DMA((P,)),
            pltpu.SemaphoreType.DMA((P,)),
            pltpu.SemaphoreType.DMA((P,)),
            pltpu.SemaphoreType.DMA((P,)),
            pltpu.SemaphoreType.DMA((G,)),
            pltpu.SemaphoreType.DMA((G,)),
        ],
        compiler_params=pltpu.CompilerParams(collective_id=0),
    )(dy, W)
